# Initial kernel scaffold; baseline (speedup 1.0000x reference)
import jax
import jax.numpy as jnp
from jax.experimental import pallas as pl


def kernel(x, col_idxs):
    raise NotImplementedError("write your pallas kernel here")


# speedup vs baseline: 1.5595x; 1.5595x over previous
import jax
import jax.numpy as jnp
from jax.experimental import pallas as pl
from jax.experimental.pallas import tpu as pltpu

_BR = 512


def _body(ci_ref, x_ref, o_ref):
    xv = x_ref[...]
    n = xv.shape[1]
    k = ci_ref.shape[1]
    ids = jax.lax.broadcasted_iota(jnp.int32, (k, n), 1)
    hit = ids == ci_ref[...].reshape(k, 1)
    mask = jnp.any(hit, axis=0, keepdims=True)
    binar = (xv >= 0.0).astype(xv.dtype)
    o_ref[...] = jnp.where(mask, binar, xv)


def kernel(x, col_idxs):
    m, n = x.shape
    k = col_idxs.shape[0]
    ci = col_idxs.reshape(1, k)
    grid = (m // _BR,)
    return pl.pallas_call(
        _body,
        grid=grid,
        in_specs=[
            pl.BlockSpec((1, k), lambda i: (0, 0)),
            pl.BlockSpec((_BR, n), lambda i: (i, 0)),
        ],
        out_specs=pl.BlockSpec((_BR, n), lambda i: (i, 0)),
        out_shape=jax.ShapeDtypeStruct((m, n), x.dtype),
        compiler_params=pltpu.CompilerParams(
            dimension_semantics=("arbitrary",),
        ),
    )(ci, x)


# speedup vs baseline: 3.6145x; 2.3177x over previous
import jax
import jax.numpy as jnp
from jax.experimental import pallas as pl
from jax.experimental.pallas import tpu as pltpu

_BR = 512


def _body(mask_ref, x_ref, o_ref):
    xv = x_ref[...]
    mask = mask_ref[...] != 0
    binar = (xv >= 0.0).astype(xv.dtype)
    o_ref[...] = jnp.where(mask, binar, xv)


def kernel(x, col_idxs):
    m, n = x.shape
    mask = jnp.zeros((1, n), jnp.int32).at[0, col_idxs].set(1)
    grid = (m // _BR,)
    return pl.pallas_call(
        _body,
        grid=grid,
        in_specs=[
            pl.BlockSpec((1, n), lambda i: (0, 0)),
            pl.BlockSpec((_BR, n), lambda i: (i, 0)),
        ],
        out_specs=pl.BlockSpec((_BR, n), lambda i: (i, 0)),
        out_shape=jax.ShapeDtypeStruct((m, n), x.dtype),
        compiler_params=pltpu.CompilerParams(
            dimension_semantics=("parallel",),
        ),
    )(mask, x)


# speedup vs baseline: 3.6899x; 1.0209x over previous
import jax
import jax.numpy as jnp
from jax.experimental import pallas as pl
from jax.experimental.pallas import tpu as pltpu

_BR = 1024


def _body(mask_ref, x_ref, o_ref):
    xv = x_ref[...]
    mask = mask_ref[...] != 0
    binar = (xv >= 0.0).astype(xv.dtype)
    o_ref[...] = jnp.where(mask, binar, xv)


def kernel(x, col_idxs):
    m, n = x.shape
    mask = jnp.zeros((1, n), jnp.int32).at[0, col_idxs].set(1)
    grid = (m // _BR,)
    return pl.pallas_call(
        _body,
        grid=grid,
        in_specs=[
            pl.BlockSpec((1, n), lambda i: (0, 0)),
            pl.BlockSpec((_BR, n), lambda i: (i, 0)),
        ],
        out_specs=pl.BlockSpec((_BR, n), lambda i: (i, 0)),
        out_shape=jax.ShapeDtypeStruct((m, n), x.dtype),
        compiler_params=pltpu.CompilerParams(
            dimension_semantics=("parallel",),
        ),
    )(mask, x)


# speedup vs baseline: 3.9387x; 1.0674x over previous
import jax
import jax.numpy as jnp
from jax.experimental import pallas as pl
from jax.experimental.pallas import tpu as pltpu

_BR = 1536


def _body(mask_ref, x_ref, o_ref):
    xv = x_ref[...]
    mask = mask_ref[...] != 0
    binar = (xv >= 0.0).astype(xv.dtype)
    o_ref[...] = jnp.where(mask, binar, xv)


def kernel(x, col_idxs):
    m, n = x.shape
    mask = jnp.zeros((1, n), jnp.int32).at[0, col_idxs].set(1)
    grid = (m // _BR,)
    return pl.pallas_call(
        _body,
        grid=grid,
        in_specs=[
            pl.BlockSpec((1, n), lambda i: (0, 0)),
            pl.BlockSpec((_BR, n), lambda i: (i, 0)),
        ],
        out_specs=pl.BlockSpec((_BR, n), lambda i: (i, 0)),
        out_shape=jax.ShapeDtypeStruct((m, n), x.dtype),
        compiler_params=pltpu.CompilerParams(
            dimension_semantics=("parallel",),
        ),
    )(mask, x)
